# baseline (speedup 1.0000x reference)
import functools
import jax
import jax.numpy as jnp
from jax import lax
from jax.experimental import pallas as pl
from jax.experimental.pallas import tpu as pltpu
from jax.experimental.pallas import tpu_sc as plsc

D_MODEL = 64
SCALE = 8.0

B_TOTAL = 4096 * 200
NUM_WORKERS = 32
B_PER_W = B_TOTAL // NUM_WORKERS
CHUNK = 128
NCHUNK = B_PER_W // CHUNK
LANES = 16
VECS_PER_ROW = D_MODEL // LANES
NBUF = 4
NGROUP = NCHUNK // NBUF

_mesh = plsc.VectorSubcoreMesh(core_axis_name="c", subcore_axis_name="s")


@functools.partial(
    pl.kernel,
    mesh=_mesh,
    out_type=jax.ShapeDtypeStruct((B_TOTAL, D_MODEL), jnp.float32),
    scratch_types=[
        pltpu.VMEM((NCHUNK, CHUNK), jnp.int32),
        pltpu.VMEM((NBUF, CHUNK, D_MODEL), jnp.float32),
        pltpu.VMEM((NBUF, CHUNK, D_MODEL), jnp.float32),
        pltpu.SemaphoreType.DMA((NBUF,)),
        pltpu.SemaphoreType.DMA((NBUF,)),
    ],
    compiler_params=pltpu.CompilerParams(use_tc_tiling_on_sc=False),
)
def _embed_sc(x_hbm, table_hbm, out_hbm, idx_v, rows_v, obuf_v, gsem, osem):
    wid = lax.axis_index("s") * 2 + lax.axis_index("c")
    out_base = wid * B_PER_W
    pltpu.sync_copy(x_hbm.at[pl.ds(wid * NCHUNK, NCHUNK)], idx_v)

    for b in range(NBUF):
        pltpu.async_copy(table_hbm.at[idx_v.at[b]], rows_v.at[b], gsem.at[b])

    def group_body(g, _):
        j0 = g * NBUF
        for b in range(NBUF):
            j = j0 + b
            pltpu.make_async_copy(
                table_hbm.at[idx_v.at[j]], rows_v.at[b], gsem.at[b]
            ).wait()

            @pl.when(j >= NBUF)
            def _():
                pltpu.make_async_copy(
                    obuf_v.at[b],
                    out_hbm.at[pl.ds(out_base + (j - NBUF) * CHUNK, CHUNK)],
                    osem.at[b],
                ).wait()

            def scale_body(r8, _):
                base = r8 * 8
                for rr in range(8):
                    for c in range(VECS_PER_ROW):
                        sl = pl.ds(c * LANES, LANES)
                        obuf_v[b, base + rr, sl] = rows_v[b, base + rr, sl] * SCALE
                return 0

            lax.fori_loop(0, CHUNK // 8, scale_body, 0)

            @pl.when(j + NBUF < NCHUNK)
            def _():
                pltpu.async_copy(
                    table_hbm.at[idx_v.at[j + NBUF]], rows_v.at[b], gsem.at[b]
                )

            pltpu.async_copy(
                obuf_v.at[b],
                out_hbm.at[pl.ds(out_base + j * CHUNK, CHUNK)],
                osem.at[b],
            )
        return 0

    lax.fori_loop(0, NGROUP, group_body, 0)

    for b in range(NBUF):
        j = NCHUNK - NBUF + b
        pltpu.make_async_copy(
            obuf_v.at[b],
            out_hbm.at[pl.ds(out_base + j * CHUNK, CHUNK)],
            osem.at[b],
        ).wait()


def kernel(x, table):
    xf = x.reshape(NUM_WORKERS * NCHUNK, CHUNK)
    out = _embed_sc(xf, table)
    return out.reshape(4096, 200, D_MODEL)


# speedup vs baseline: 1.0006x; 1.0006x over previous
import functools
import jax
import jax.numpy as jnp
from jax import lax
from jax.experimental import pallas as pl
from jax.experimental.pallas import tpu as pltpu
from jax.experimental.pallas import tpu_sc as plsc

D_MODEL = 64
SCALE = 8.0

NROWS = 4096
ROW_W = 200
NUM_WORKERS = 32
ROWS_PER_W = NROWS // NUM_WORKERS
LANES = 16
VECS_PER_ROW = D_MODEL // LANES
NBUF = 4
NGROUP = ROWS_PER_W // NBUF

_mesh = plsc.VectorSubcoreMesh(core_axis_name="c", subcore_axis_name="s")


@functools.partial(
    pl.kernel,
    mesh=_mesh,
    out_type=jax.ShapeDtypeStruct((NROWS, ROW_W, D_MODEL), jnp.float32),
    scratch_types=[
        pltpu.VMEM((ROWS_PER_W, ROW_W), jnp.int32),
        pltpu.VMEM((NBUF, ROW_W, D_MODEL), jnp.float32),
        pltpu.VMEM((NBUF, ROW_W, D_MODEL), jnp.float32),
        pltpu.SemaphoreType.DMA((NBUF,)),
        pltpu.SemaphoreType.DMA((NBUF,)),
    ],
    compiler_params=pltpu.CompilerParams(use_tc_tiling_on_sc=False),
)
def _embed_sc(x_hbm, table_hbm, out_hbm, idx_v, rows_v, obuf_v, gsem, osem):
    wid = lax.axis_index("s") * 2 + lax.axis_index("c")
    row_base = wid * ROWS_PER_W
    pltpu.sync_copy(x_hbm.at[pl.ds(row_base, ROWS_PER_W)], idx_v)

    for b in range(NBUF):
        pltpu.async_copy(table_hbm.at[idx_v.at[b]], rows_v.at[b], gsem.at[b])

    def group_body(g, _):
        j0 = g * NBUF
        for b in range(NBUF):
            j = j0 + b

            pltpu.make_async_copy(
                table_hbm.at[idx_v.at[j]], rows_v.at[b], gsem.at[b]
            ).wait()

            @pl.when(j >= NBUF)
            def _():
                pltpu.make_async_copy(
                    obuf_v.at[b], out_hbm.at[j - NBUF + row_base], osem.at[b]
                ).wait()

            def scale_body(r8, _):
                base = r8 * 8
                for rr in range(8):
                    for c in range(VECS_PER_ROW):
                        sl = pl.ds(c * LANES, LANES)
                        obuf_v[b, base + rr, sl] = (
                            rows_v[b, base + rr, sl] * SCALE
                        )
                return 0

            lax.fori_loop(0, ROW_W // 8, scale_body, 0)

            @pl.when(j + NBUF < ROWS_PER_W)
            def _():
                pltpu.async_copy(
                    table_hbm.at[idx_v.at[j + NBUF]], rows_v.at[b], gsem.at[b]
                )

            pltpu.async_copy(obuf_v.at[b], out_hbm.at[j + row_base], osem.at[b])
        return 0

    lax.fori_loop(0, NGROUP, group_body, 0)

    for b in range(NBUF):
        j = ROWS_PER_W - NBUF + b
        pltpu.make_async_copy(
            obuf_v.at[b], out_hbm.at[j + row_base], osem.at[b]
        ).wait()


def kernel(x, table):
    return _embed_sc(x, table)


# speedup vs baseline: 1.1097x; 1.1090x over previous
import functools
import jax
import jax.numpy as jnp
from jax import lax
from jax.experimental import pallas as pl
from jax.experimental.pallas import tpu as pltpu
from jax.experimental.pallas import tpu_sc as plsc

D_MODEL = 64
SCALE = 8.0

BATCH = 4096
NPOS = 200
NUM_WORKERS = 32
BL = 128
NP8 = NPOS // 8
NF8 = D_MODEL // 8
LANES = 16
NBUF = 4
NGROUP = NPOS // NBUF

_mesh = plsc.VectorSubcoreMesh(core_axis_name="c", subcore_axis_name="s")


@functools.partial(
    pl.kernel,
    mesh=_mesh,
    out_type=jax.ShapeDtypeStruct((NPOS, NF8, NUM_WORKERS, 8, BL), jnp.float32),
    scratch_types=[
        pltpu.VMEM((NPOS, BL), jnp.int32),
        pltpu.VMEM((NBUF, BL, D_MODEL), jnp.float32),
        pltpu.VMEM((NBUF, D_MODEL, 136), jnp.float32),
        pltpu.SemaphoreType.DMA,
        pltpu.SemaphoreType.DMA((NBUF,)),
        pltpu.SemaphoreType.DMA((NBUF,)),
    ],
    compiler_params=pltpu.CompilerParams(
        use_tc_tiling_on_sc=False, needs_layout_passes=False
    ),
)
def _embed_sc(x_hbm, table_hbm, out_hbm, idx_v, rows_v, obuf_v, isem, gsem, osem):
    wid = lax.axis_index("s") * 2 + lax.axis_index("c")

    for p8 in range(NP8):
        pltpu.async_copy(x_hbm.at[p8, wid], idx_v.at[pl.ds(p8 * 8, 8)], isem)
    for p8 in range(NP8):
        pltpu.make_async_copy(
            x_hbm.at[p8, wid], idx_v.at[pl.ds(p8 * 8, 8)], isem
        ).wait()

    for b in range(NBUF):
        pltpu.async_copy(table_hbm.at[idx_v.at[b]], rows_v.at[b], gsem.at[b])

    lane = lax.iota(jnp.int32, 16)

    def group_body(g, _):
        j0 = g * NBUF
        for b in range(NBUF):
            j = j0 + b

            pltpu.make_async_copy(
                table_hbm.at[idx_v.at[j]], rows_v.at[b], gsem.at[b]
            ).wait()

            @pl.when(j >= NBUF)
            def _():
                for f8 in range(NF8):
                    pltpu.make_async_copy(
                        obuf_v.at[b, pl.ds(f8 * 8, 8), pl.ds(0, BL)],
                        out_hbm.at[j - NBUF, f8, wid],
                        osem.at[b],
                    ).wait()

            def trans_body(r4, _):
                r = r4 * 4
                for rr in range(4):
                    ridx = jnp.full((16,), 0, jnp.int32) + (r + rr)
                    for c in range(D_MODEL // LANES):
                        sl = pl.ds(c * LANES, LANES)
                        v = rows_v[b, r + rr, sl] * SCALE
                        plsc.store_scatter(
                            obuf_v.at[b], [lane + c * LANES, ridx], v
                        )
                return 0

            lax.fori_loop(0, BL // 4, trans_body, 0)

            @pl.when(j + NBUF < NPOS)
            def _():
                pltpu.async_copy(
                    table_hbm.at[idx_v.at[j + NBUF]], rows_v.at[b], gsem.at[b]
                )

            for f8 in range(NF8):
                pltpu.async_copy(
                    obuf_v.at[b, pl.ds(f8 * 8, 8), pl.ds(0, BL)],
                    out_hbm.at[j, f8, wid],
                    osem.at[b],
                )
        return 0

    lax.fori_loop(0, NGROUP, group_body, 0)

    for b in range(NBUF):
        j = NPOS - NBUF + b
        for f8 in range(NF8):
            pltpu.make_async_copy(
                obuf_v.at[b, pl.ds(f8 * 8, 8), pl.ds(0, BL)],
                out_hbm.at[j, f8, wid],
                osem.at[b],
            ).wait()


def kernel(x, table):
    xv = x.reshape(NUM_WORKERS, BL, NP8, 8).transpose(2, 0, 3, 1)
    out_phys = _embed_sc(xv, table)
    out = out_phys.transpose(2, 4, 0, 1, 3)
    return out.reshape(BATCH, NPOS, D_MODEL)
